# baseline (speedup 1.0000x reference)
import jax
import jax.numpy as jnp
from jax.experimental import pallas as pl
from jax.experimental.pallas import tpu as pltpu

IN_F = 512
OUT_F = 512
TM = 256


def _gcn_kernel(x0_ref, xs_ref, adj_ref, w_ref, b_ref, out_ref,
                support_ref):
    i = pl.program_id(0)
    m = pl.program_id(1)
    nb = pl.num_programs(0)

    wb = w_ref[...].astype(jnp.bfloat16)

    @pl.when((i == 0) & (m == 0))
    def _():
        xb = x0_ref[0].astype(jnp.bfloat16)
        support_ref[0] = jnp.dot(
            xb, wb, preferred_element_type=jnp.float32
        ).astype(jnp.bfloat16)

    @pl.when(i < nb - 1)
    def _():
        xs = xs_ref[0].astype(jnp.bfloat16)
        support_ref[(i + 1) % 2, pl.ds(m * TM, TM)] = jnp.dot(
            xs, wb, preferred_element_type=jnp.float32
        ).astype(jnp.bfloat16)

    a = adj_ref[0].astype(jnp.bfloat16)
    acc = jnp.dot(a, support_ref[i % 2], preferred_element_type=jnp.float32)
    out_ref[0] = acc + b_ref[...]


def kernel(input, adj, W, b):
    B, N, _ = input.shape
    grid = (B, N // TM)
    b2d = b.reshape(1, OUT_F)

    def xs_index(i, m):
        nxt = jnp.minimum(i + 1, B - 1)
        return (nxt, jnp.where(i + 1 < B, m, 0), 0)

    return pl.pallas_call(
        _gcn_kernel,
        grid=grid,
        in_specs=[
            pl.BlockSpec((1, N, IN_F), lambda i, m: (0, 0, 0)),
            pl.BlockSpec((1, TM, IN_F), xs_index),
            pl.BlockSpec((1, TM, N), lambda i, m: (i, m, 0)),
            pl.BlockSpec((IN_F, OUT_F), lambda i, m: (0, 0)),
            pl.BlockSpec((1, OUT_F), lambda i, m: (0, 0)),
        ],
        out_specs=pl.BlockSpec((1, TM, OUT_F), lambda i, m: (i, m, 0)),
        out_shape=jax.ShapeDtypeStruct((B, N, OUT_F), jnp.float32),
        scratch_shapes=[pltpu.VMEM((2, N, OUT_F), jnp.bfloat16)],
        compiler_params=pltpu.CompilerParams(
            dimension_semantics=("arbitrary", "arbitrary"),
        ),
    )(input, input, adj, W, b2d)


# speedup vs baseline: 1.1924x; 1.1924x over previous
import jax
import jax.numpy as jnp
from jax.experimental import pallas as pl
from jax.experimental.pallas import tpu as pltpu

IN_F = 512
OUT_F = 512
TM = 512


def _gcn_kernel(x0_ref, xs_ref, adj_ref, w_ref, b_ref, out_ref,
                support_ref):
    i = pl.program_id(0)
    m = pl.program_id(1)
    nb = pl.num_programs(0)

    wb = w_ref[...].astype(jnp.bfloat16)

    @pl.when((i == 0) & (m == 0))
    def _():
        xb = x0_ref[0].astype(jnp.bfloat16)
        support_ref[0] = jnp.dot(
            xb, wb, preferred_element_type=jnp.float32
        )

    @pl.when(i < nb - 1)
    def _():
        xs = xs_ref[0].astype(jnp.bfloat16)
        support_ref[(i + 1) % 2, pl.ds(m * TM, TM)] = jnp.dot(
            xs, wb, preferred_element_type=jnp.float32
        )

    acc = jax.lax.dot_general(
        adj_ref[0], support_ref[i % 2],
        (((1,), (0,)), ((), ())),
        precision=jax.lax.Precision.DEFAULT,
        preferred_element_type=jnp.float32)
    out_ref[0] = acc + b_ref[...]


def kernel(input, adj, W, b):
    B, N, _ = input.shape
    grid = (B, N // TM)
    b2d = b.reshape(1, OUT_F)

    def xs_index(i, m):
        nxt = jnp.minimum(i + 1, B - 1)
        return (nxt, jnp.where(i + 1 < B, m, 0), 0)

    return pl.pallas_call(
        _gcn_kernel,
        grid=grid,
        in_specs=[
            pl.BlockSpec((1, N, IN_F), lambda i, m: (0, 0, 0)),
            pl.BlockSpec((1, TM, IN_F), xs_index),
            pl.BlockSpec((1, TM, N), lambda i, m: (i, m, 0)),
            pl.BlockSpec((IN_F, OUT_F), lambda i, m: (0, 0)),
            pl.BlockSpec((1, OUT_F), lambda i, m: (0, 0)),
        ],
        out_specs=pl.BlockSpec((1, TM, OUT_F), lambda i, m: (i, m, 0)),
        out_shape=jax.ShapeDtypeStruct((B, N, OUT_F), jnp.float32),
        scratch_shapes=[pltpu.VMEM((2, N, OUT_F), jnp.float32)],
        compiler_params=pltpu.CompilerParams(
            dimension_semantics=("arbitrary", "arbitrary"),
        ),
    )(input, input, adj, W, b2d)


# speedup vs baseline: 1.2001x; 1.0065x over previous
import jax
import jax.numpy as jnp
from jax.experimental import pallas as pl
from jax.experimental.pallas import tpu as pltpu

IN_F = 512
OUT_F = 512
TM = 512


def _gcn_kernel(x0_ref, xs_ref, adj_ref, w_ref, b_ref, out_ref,
                support_ref):
    i = pl.program_id(0)
    m = pl.program_id(1)
    nb = pl.num_programs(0)

    @pl.when((i == 0) & (m == 0))
    def _():
        support_ref[0] = jax.lax.dot_general(
            x0_ref[0], w_ref[...],
            (((1,), (0,)), ((), ())),
            precision=jax.lax.Precision.DEFAULT,
            preferred_element_type=jnp.float32)

    @pl.when(i < nb - 1)
    def _():
        support_ref[(i + 1) % 2, pl.ds(m * TM, TM)] = jax.lax.dot_general(
            xs_ref[0], w_ref[...],
            (((1,), (0,)), ((), ())),
            precision=jax.lax.Precision.DEFAULT,
            preferred_element_type=jnp.float32)

    acc = jax.lax.dot_general(
        adj_ref[0], support_ref[i % 2],
        (((1,), (0,)), ((), ())),
        precision=jax.lax.Precision.DEFAULT,
        preferred_element_type=jnp.float32)
    out_ref[0] = acc + b_ref[...]


def kernel(input, adj, W, b):
    B, N, _ = input.shape
    grid = (B, N // TM)
    b2d = b.reshape(1, OUT_F)

    def xs_index(i, m):
        nxt = jnp.minimum(i + 1, B - 1)
        return (nxt, jnp.where(i + 1 < B, m, 0), 0)

    return pl.pallas_call(
        _gcn_kernel,
        grid=grid,
        in_specs=[
            pl.BlockSpec((1, N, IN_F), lambda i, m: (0, 0, 0)),
            pl.BlockSpec((1, TM, IN_F), xs_index),
            pl.BlockSpec((1, TM, N), lambda i, m: (i, m, 0)),
            pl.BlockSpec((IN_F, OUT_F), lambda i, m: (0, 0)),
            pl.BlockSpec((1, OUT_F), lambda i, m: (0, 0)),
        ],
        out_specs=pl.BlockSpec((1, TM, OUT_F), lambda i, m: (i, m, 0)),
        out_shape=jax.ShapeDtypeStruct((B, N, OUT_F), jnp.float32),
        scratch_shapes=[pltpu.VMEM((2, N, OUT_F), jnp.float32)],
        compiler_params=pltpu.CompilerParams(
            dimension_semantics=("arbitrary", "arbitrary"),
        ),
    )(input, input, adj, W, b2d)


# speedup vs baseline: 1.2054x; 1.0044x over previous
import jax
import jax.numpy as jnp
from jax.experimental import pallas as pl
from jax.experimental.pallas import tpu as pltpu

IN_F = 512
OUT_F = 512
TM = 512


def _gcn_kernel(x_ref, adj_ref, w_ref, b_ref, out_ref):
    t = jax.lax.dot_general(
        adj_ref[0], x_ref[0],
        (((1,), (0,)), ((), ())),
        precision=jax.lax.Precision.DEFAULT,
        preferred_element_type=jnp.float32)
    acc = jax.lax.dot_general(
        t, w_ref[...],
        (((1,), (0,)), ((), ())),
        precision=jax.lax.Precision.DEFAULT,
        preferred_element_type=jnp.float32)
    out_ref[0] = acc + b_ref[...]


def kernel(input, adj, W, b):
    B, N, _ = input.shape
    grid = (B, N // TM)
    b2d = b.reshape(1, OUT_F)
    return pl.pallas_call(
        _gcn_kernel,
        grid=grid,
        in_specs=[
            pl.BlockSpec((1, N, IN_F), lambda i, m: (i, 0, 0)),
            pl.BlockSpec((1, TM, N), lambda i, m: (i, m, 0)),
            pl.BlockSpec((IN_F, OUT_F), lambda i, m: (0, 0)),
            pl.BlockSpec((1, OUT_F), lambda i, m: (0, 0)),
        ],
        out_specs=pl.BlockSpec((1, TM, OUT_F), lambda i, m: (i, m, 0)),
        out_shape=jax.ShapeDtypeStruct((B, N, OUT_F), jnp.float32),
        compiler_params=pltpu.CompilerParams(
            dimension_semantics=("arbitrary", "arbitrary"),
        ),
    )(input, adj, W, b2d)
